# Initial kernel scaffold; baseline (speedup 1.0000x reference)
import jax
import jax.numpy as jnp
from jax.experimental import pallas as pl


def kernel(atoms, pos, edge_index, batch, emb, ms_W1, ms_b1, ms_W2, ms_b2, vg_Wq, vg_Wk, pred_W1, pred_b1, pred_W2, pred_b2):
    raise NotImplementedError("write your pallas kernel here")


# speedup vs baseline: 3.7447x; 3.7447x over previous
import jax
import jax.numpy as jnp
from jax.experimental import pallas as pl

N = 10000
NP = 10240
EMB = 128
SDIM = 16
HID = 32
L = 5
VOCABP = 128
G = 64

_HI = jax.lax.Precision.HIGHEST


def _dot_bf(a, b):
    return jnp.dot(a.astype(jnp.bfloat16), b.astype(jnp.bfloat16),
                   preferred_element_type=jnp.float32)


def _hmp_kernel(atoms_ref, batch_ref, emb_ref, w1_ref, b1_ref, w2_ref,
                b2_ref, pw1_ref, pb1_ref, pw2_ref, pb2_ref, out_ref):
    f32 = jnp.float32
    av = atoms_ref[...]
    iota_v = jax.lax.broadcasted_iota(jnp.int32, (1, VOCABP), 1)
    onehot_a = (av == iota_v).astype(f32)
    h = jnp.dot(onehot_a, emb_ref[...], precision=_HI,
                preferred_element_type=f32)

    rows = jax.lax.broadcasted_iota(jnp.int32, (NP, 1), 0)
    validf = (rows < N).astype(f32)

    for i in range(L):
        hs = h[:, :SDIM]
        z = jax.nn.relu(_dot_bf(hs, w1_ref[i]) + b1_ref[i])
        t = _dot_bf(z, w2_ref[i]) + b2_ref[pl.ds(i, 1), :]
        m = jax.nn.sigmoid(t)
        mask = m > 0.5
        k = jnp.sum(jnp.where(mask, validf, 0.0))
        maskf = mask.astype(f32)
        h_master = h * maskf
        expanded = jnp.where(mask, h_master, jnp.zeros_like(h))
        blended = (1.0 - m) * h + m * expanded
        h = jnp.where(k > 1.5, blended, h)

    iota_g = jax.lax.broadcasted_iota(jnp.int32, (1, G), 1)
    onehot_b = (batch_ref[...] == iota_g).astype(f32)
    pooled = jax.lax.dot_general(onehot_b, h, (((0,), (0,)), ((), ())),
                                 precision=_HI,
                                 preferred_element_type=f32)
    hid = jax.nn.relu(_dot_bf(pooled, pw1_ref[...]) + pb1_ref[...])
    out_ref[...] = _dot_bf(hid, pw2_ref[...]) + pb2_ref[...]


def kernel(atoms, pos, edge_index, batch, emb, ms_W1, ms_b1, ms_W2, ms_b2,
           vg_Wq, vg_Wk, pred_W1, pred_b1, pred_W2, pred_b2):
    del pos, edge_index, vg_Wq, vg_Wk
    atoms2 = jnp.pad(atoms.astype(jnp.int32), (0, NP - N),
                     constant_values=VOCABP - 1).reshape(NP, 1)
    batch2 = jnp.pad(batch.astype(jnp.int32), (0, NP - N),
                     constant_values=G - 1).reshape(NP, 1)
    emb_p = jnp.pad(emb, ((0, VOCABP - emb.shape[0]), (0, 0)))
    b1r = ms_b1.reshape(L, 1, HID)
    pb1r = pred_b1.reshape(1, EMB // 2)
    pb2r = pred_b2.reshape(1, 1)
    return pl.pallas_call(
        _hmp_kernel,
        out_shape=jax.ShapeDtypeStruct((G, 1), jnp.float32),
    )(atoms2, batch2, emb_p, ms_W1, b1r, ms_W2, ms_b2, pred_W1, pb1r,
      pred_W2, pb2r)


# speedup vs baseline: 11.4697x; 3.0629x over previous
import jax
import jax.numpy as jnp
from jax.experimental import pallas as pl

N = 10000
NP = 10240
EMB = 128
SDIM = 16
HID = 32
L = 5
VOCABP = 128
G = 64

_HI = jax.lax.Precision.HIGHEST
_BF = jnp.bfloat16
_F32 = jnp.float32


def _bf_split3(x):
    p1 = x.astype(_BF)
    r1 = x - p1.astype(_F32)
    p2 = r1.astype(_BF)
    p3 = (r1 - p2.astype(_F32)).astype(_BF)
    return p1, p2, p3


def _hmp_kernel(atoms_ref, batch_ref, embT_ref, w1T_ref, b1_ref, w2_ref,
                b2_ref, pw1T_ref, pb1_ref, pw2T_ref, pb2_ref, out_ref):
    iota_v = jax.lax.broadcasted_iota(jnp.int32, (VOCABP, 1), 0)
    onehotT = (atoms_ref[...] == iota_v).astype(_BF)

    embT16 = embT_ref[:SDIM, :]
    e1, e2, e3 = _bf_split3(embT16)
    def _g(p):
        return jnp.dot(p, onehotT, preferred_element_type=_F32)
    hsT = (_g(e1) + _g(e2)) + _g(e3)

    lanes = jax.lax.broadcasted_iota(jnp.int32, (1, NP), 1)
    validf = (lanes < N).astype(_F32)

    s = jnp.ones((1, NP), _F32)
    for i in range(L):
        zT = jax.nn.relu(
            jnp.dot(w1T_ref[i].astype(_BF), hsT.astype(_BF),
                    preferred_element_type=_F32) + b1_ref[i])
        prod = (zT.astype(_BF).astype(_F32)
                * w2_ref[i].astype(_BF).astype(_F32))
        t = (jnp.sum(prod, axis=0, keepdims=True)
             + b2_ref[pl.ds(i, 1), :])
        m = jax.nn.sigmoid(t)
        mask = m > 0.5
        k = jnp.sum(jnp.where(mask, validf, 0.0))
        cond = k > 1.5
        maskf = mask.astype(_F32)
        h_master = hsT * maskf
        expanded = jnp.where(mask, h_master, jnp.zeros_like(hsT))
        blended = (1.0 - m) * hsT + m * expanded
        hsT = jnp.where(cond, blended, hsT)
        lscale = jnp.where(mask, 1.0, 1.0 - m)
        s = jnp.where(cond, s * lscale, s)

    iota_g = jax.lax.broadcasted_iota(jnp.int32, (1, G), 1)
    onehot_b = (batch_ref[...] == iota_g).astype(_BF)
    s1, s2, s3 = _bf_split3(s)
    def _p(sk):
        wa = onehotT * sk
        return jnp.dot(wa, onehot_b, preferred_element_type=_F32)
    wgvT = (_p(s1) + _p(s2)) + _p(s3)
    pooledT = jnp.dot(embT_ref[...], wgvT, precision=_HI,
                      preferred_element_type=_F32)

    hidT = jax.nn.relu(
        jnp.dot(pw1T_ref[...].astype(_BF), pooledT.astype(_BF),
                preferred_element_type=_F32) + pb1_ref[...])
    out_ref[...] = (jnp.dot(pw2T_ref[...].astype(_BF), hidT.astype(_BF),
                            preferred_element_type=_F32)
                    + pb2_ref[...])


def kernel(atoms, pos, edge_index, batch, emb, ms_W1, ms_b1, ms_W2, ms_b2,
           vg_Wq, vg_Wk, pred_W1, pred_b1, pred_W2, pred_b2):
    del pos, edge_index, vg_Wq, vg_Wk
    atoms_row = jnp.pad(atoms.astype(jnp.int32), (0, NP - N),
                        constant_values=VOCABP - 1).reshape(1, NP)
    batch_col = jnp.pad(batch.astype(jnp.int32), (0, NP - N),
                        constant_values=G - 1).reshape(NP, 1)
    embT = jnp.pad(emb, ((0, VOCABP - emb.shape[0]), (0, 0))).T
    w1T = ms_W1.transpose(0, 2, 1)
    b1r = ms_b1.reshape(L, HID, 1)
    pw1T = pred_W1.T
    pb1r = pred_b1.reshape(EMB // 2, 1)
    pw2T = pred_W2.reshape(1, EMB // 2)
    pb2r = pred_b2.reshape(1, 1)
    outT = pl.pallas_call(
        _hmp_kernel,
        out_shape=jax.ShapeDtypeStruct((1, G), jnp.float32),
    )(atoms_row, batch_col, embT, w1T, b1r, ms_W2, ms_b2, pw1T, pb1r,
      pw2T, pb2r)
    return outT.reshape(G, 1)
